# baseline (speedup 1.0000x reference)
import functools

import jax
import jax.numpy as jnp
from jax import lax
from jax.experimental import pallas as pl
from jax.experimental.pallas import tpu as pltpu
from jax.experimental.pallas import tpu_sc as plsc

_N = 10000
_E = 160000
_D = 128
_H = 64

_NC = 2
_NS = 16
_DH = _D // _NC

_CHUNK = 125
_CPT = 80
_NBUF = 4
_RPS = _N // _NS
_CW = 16

_TILE_ROWS = 2000


def _sc_segment_sums(tbl_iu, tbl_ui, src_iu, dst_iu, src_ui, dst_ui,
                     zrow, zcnt, ones_h):
    mesh = plsc.VectorSubcoreMesh(core_axis_name="c", subcore_axis_name="s")

    @functools.partial(
        pl.kernel,
        out_type=(
            jax.ShapeDtypeStruct((_NC, _N, _DH), jnp.float32),
            jax.ShapeDtypeStruct((_N, _CW), jnp.float32),
            jax.ShapeDtypeStruct((_NC, _N, _DH), jnp.float32),
            jax.ShapeDtypeStruct((_N, _CW), jnp.float32),
        ),
        mesh=mesh,
        compiler_params=pltpu.CompilerParams(use_tc_tiling_on_sc=False),
        scratch_types=[
            pltpu.VMEM((_CPT, _CHUNK), jnp.int32),
            pltpu.VMEM((_CPT, _CHUNK), jnp.int32),
            pltpu.VMEM((_CHUNK, _DH), jnp.float32),
            pltpu.VMEM((_CHUNK, _DH), jnp.float32),
            pltpu.VMEM((_CHUNK, _DH), jnp.float32),
            pltpu.VMEM((_CHUNK, _DH), jnp.float32),
            pltpu.VMEM((_CHUNK, _CW), jnp.float32),
            pltpu.VMEM_SHARED((_N, _DH), jnp.float32),
            pltpu.VMEM_SHARED((_N, _CW), jnp.float32),
            pltpu.SemaphoreType.DMA,
            pltpu.SemaphoreType.DMA,
            pltpu.SemaphoreType.DMA,
            pltpu.SemaphoreType.DMA,
            pltpu.SemaphoreType.DMA,
            pltpu.SemaphoreType.DMA,
            pltpu.SemaphoreType.DMA,
            pltpu.SemaphoreType.DMA,
            pltpu.SemaphoreType.DMA,
        ],
    )
    def k(tbl_iu_h, tbl_ui_h, src_iu_h, dst_iu_h, src_ui_h, dst_ui_h,
          zrow_h, zcnt_h, ones_hh,
          out_su, out_cu, out_si, out_ci,
          srcv, dstv, b0, b1, b2, b3, onesv, acc, accc,
          sg0, sg1, sg2, sg3, ss0, ss1, ss2, ss3, sc):
        cid = lax.axis_index("c")
        sid = lax.axis_index("s")
        r0 = sid * _RPS
        bufs = ((b0, sg0, ss0), (b1, sg1, ss1), (b2, sg2, ss2), (b3, sg3, ss3))

        pltpu.sync_copy(ones_hh, onesv)

        def zero_acc():
            pltpu.sync_copy(zrow_h, acc.at[pl.ds(r0, _RPS)])
            pltpu.sync_copy(zcnt_h, accc.at[pl.ds(r0, _RPS)])

        def run_type(src_h, dst_h, table_h, out_s, out_c, cnt_core):
            pltpu.sync_copy(src_h.at[cid, sid], srcv)
            pltpu.sync_copy(dst_h.at[sid], dstv)
            do_cnt = cid == cnt_core

            def fire_gather(t, buf, sem):
                pltpu.async_copy(table_h.at[srcv.at[t]], buf, sem)

            def wait_gather(t, buf, sem):
                pltpu.make_async_copy(table_h.at[srcv.at[t]], buf, sem).wait()

            def fire_scatter(t, buf, sem):
                pltpu.async_copy(buf, acc.at[dstv.at[t]], sem, add=True)

                @pl.when(do_cnt)
                def _():
                    pltpu.async_copy(onesv, accc.at[dstv.at[t]], sc, add=True)

            def wait_scatter(t, buf, sem):
                pltpu.make_async_copy(buf, acc.at[dstv.at[t]], sem).wait()

                @pl.when(do_cnt)
                def _():
                    pltpu.make_async_copy(onesv, accc.at[dstv.at[t]], sc).wait()

            fire_gather(0, b0, sg0)
            fire_gather(1, b1, sg1)

            def body(i, carry):
                for kk in range(_NBUF):
                    t = _NBUF * i + kk
                    buf, sg, ss = bufs[kk]
                    rbuf, rsg, rss = bufs[(kk + 2) % _NBUF]
                    if kk >= 2:
                        wait_scatter(t - 2, rbuf, rss)
                    else:
                        @pl.when(t >= 2)
                        def _():
                            wait_scatter(t - 2, rbuf, rss)

                    @pl.when(t + 2 < _CPT)
                    def _():
                        fire_gather(t + 2, rbuf, rsg)

                    wait_gather(t, buf, sg)
                    fire_scatter(t, buf, ss)
                return carry

            lax.fori_loop(0, _CPT // _NBUF, body, 0)
            wait_scatter(_CPT - 2, bufs[(_CPT - 2) % _NBUF][0],
                         bufs[(_CPT - 2) % _NBUF][2])
            wait_scatter(_CPT - 1, bufs[(_CPT - 1) % _NBUF][0],
                         bufs[(_CPT - 1) % _NBUF][2])
            plsc.subcore_barrier()
            pltpu.sync_copy(acc.at[pl.ds(r0, _RPS)],
                            out_s.at[cid, pl.ds(r0, _RPS)])

            @pl.when(do_cnt)
            def _():
                pltpu.sync_copy(accc.at[pl.ds(r0, _RPS)],
                                out_c.at[pl.ds(r0, _RPS)])

        zero_acc()
        plsc.subcore_barrier()
        run_type(src_iu_h, dst_iu_h, tbl_iu_h, out_su, out_cu, 0)
        zero_acc()
        plsc.subcore_barrier()
        run_type(src_ui_h, dst_ui_h, tbl_ui_h, out_si, out_ci, 1)

    return k(tbl_iu, tbl_ui, src_iu, dst_iu, src_ui, dst_ui,
             zrow, zcnt, ones_h)


def _dense_body(xu, xi, su, cu, si, ci,
                wmiu, bmiu, wmui, bmui, wp1, bp1, wp2, bp2,
                wsu, bsu, wsi, bsi, wcu, bcu, wci, bci,
                out_u, out_i):
    f32 = jnp.float32
    cuv = cu[:, 0:1]
    gu = jnp.concatenate([su[0], su[1]], -1) / jnp.maximum(cuv, 1.0)
    mu = (cuv > 0.0).astype(f32)
    xuv = xu[...]
    h = jnp.maximum(jnp.dot(xuv, wp1[...]) + bp1[...], 0.0)
    pred = jnp.dot(h, wp2[...]) + bp2[...]
    aggstd = jnp.dot(gu, wmiu[...]) + bmiu[...] * mu
    neigh = 0.5 * (aggstd + gu - pred * mu)
    selfu = jnp.dot(xuv, wsu[...]) + bsu[...]
    wcu_t = wcu[...]
    out_u[...] = jnp.maximum(
        jnp.dot(selfu, wcu_t[:_D]) + jnp.dot(neigh, wcu_t[_D:]) + bcu[...], 0.0)
    civ = ci[:, 0:1]
    gi = jnp.concatenate([si[0], si[1]], -1) / jnp.maximum(civ, 1.0)
    mi = (civ > 0.0).astype(f32)
    xiv = xi[...]
    aggi = jnp.dot(gi, wmui[...]) + bmui[...] * mi
    selfi = jnp.dot(xiv, wsi[...]) + bsi[...]
    wci_t = wci[...]
    out_i[...] = jnp.maximum(
        jnp.dot(selfi, wci_t[:_D]) + jnp.dot(aggi, wci_t[_D:]) + bci[...], 0.0)


def _dense_combine(xu, xi, su, cu, si, ci,
                   wmiu, bmiu, wmui, bmui, wp1, bp1, wp2, bp2,
                   wsu, bsu, wsi, bsi, wcu, bcu, wci, bci):
    grid = _N // _TILE_ROWS
    row = lambda width: pl.BlockSpec((_TILE_ROWS, width), lambda i: (i, 0))
    half = pl.BlockSpec((_NC, _TILE_ROWS, _DH), lambda i: (0, i, 0))
    full = lambda a, b: pl.BlockSpec((a, b), lambda i: (0, 0))
    return pl.pallas_call(
        _dense_body,
        grid=(grid,),
        in_specs=[
            row(_D), row(_D),
            half, row(_CW),
            half, row(_CW),
            full(_D, _D), full(1, _D),
            full(_D, _D), full(1, _D),
            full(_D, _H), full(1, _H),
            full(_H, _D), full(1, _D),
            full(_D, _D), full(1, _D),
            full(_D, _D), full(1, _D),
            full(2 * _D, _D), full(1, _D),
            full(2 * _D, _D), full(1, _D),
        ],
        out_specs=[row(_D), row(_D)],
        out_shape=[
            jax.ShapeDtypeStruct((_N, _D), jnp.float32),
            jax.ShapeDtypeStruct((_N, _D), jnp.float32),
        ],
    )(xu, xi, su, cu, si, ci,
      wmiu, bmiu, wmui, bmui, wp1, bp1, wp2, bp2,
      wsu, bsu, wsi, bsi, wcu, bcu, wci, bci)


def _prep_edges(ei):
    src = ei[0].astype(jnp.int32)
    dst = ei[1].astype(jnp.int32)
    src2 = jnp.stack([src, src + _N]).reshape(_NC, _NS, _CPT, _CHUNK)
    return src2, dst.reshape(_NS, _CPT, _CHUNK)


def _stack_halves(x):
    return jnp.concatenate([x[:, :_DH], x[:, _DH:]], 0)


def kernel(x_user, x_item, edge_index_item_to_user, edge_index_user_rev_item,
           W_msg_iu, b_msg_iu, W_msg_ui, b_msg_ui,
           W_pred1, b_pred1, W_pred2, b_pred2,
           W_self_user, b_self_user, W_self_item, b_self_item,
           W_comb_user, b_comb_user, W_comb_item, b_comb_item):
    x_user = x_user.astype(jnp.float32)
    x_item = x_item.astype(jnp.float32)
    src_iu, dst_iu = _prep_edges(edge_index_item_to_user)
    src_ui, dst_ui = _prep_edges(edge_index_user_rev_item)

    zrow = jnp.zeros((_RPS, _DH), jnp.float32)
    zcnt = jnp.zeros((_RPS, _CW), jnp.float32)
    ones_h = jnp.ones((_CHUNK, _CW), jnp.float32)

    su, cu, si, ci = _sc_segment_sums(
        _stack_halves(x_item), _stack_halves(x_user),
        src_iu, dst_iu, src_ui, dst_ui, zrow, zcnt, ones_h)

    out_u, out_i = _dense_combine(
        x_user, x_item, su, cu, si, ci,
        W_msg_iu.T, b_msg_iu.reshape(1, _D),
        W_msg_ui.T, b_msg_ui.reshape(1, _D),
        W_pred1.T, b_pred1.reshape(1, _H),
        W_pred2.T, b_pred2.reshape(1, _D),
        W_self_user.T, b_self_user.reshape(1, _D),
        W_self_item.T, b_self_item.reshape(1, _D),
        W_comb_user.T, b_comb_user.reshape(1, _D),
        W_comb_item.T, b_comb_item.reshape(1, _D))
    return (out_u, out_i)


# speedup vs baseline: 1.2019x; 1.2019x over previous
import functools

import jax
import jax.numpy as jnp
from jax import lax
from jax.experimental import pallas as pl
from jax.experimental.pallas import tpu as pltpu
from jax.experimental.pallas import tpu_sc as plsc

_N = 10000
_E = 160000
_D = 128
_H = 64

_NC = 2
_NS = 16
_DH = _D // _NC

_CHUNK = 125
_CPT = 80
_NBUF = 4
_RPS = _N // _NS
_CW = 16

_TILE_ROWS = 2000


def _sc_segment_sums(x_item, x_user, src_iu, dst_iu, src_ui, dst_ui,
                     zrow, zcnt, ones_h):
    mesh = plsc.VectorSubcoreMesh(core_axis_name="c", subcore_axis_name="s")

    @functools.partial(
        pl.kernel,
        out_type=(
            jax.ShapeDtypeStruct((_NC, _N, _DH), jnp.float32),
            jax.ShapeDtypeStruct((_N, _CW), jnp.float32),
            jax.ShapeDtypeStruct((_NC, _N, _DH), jnp.float32),
            jax.ShapeDtypeStruct((_N, _CW), jnp.float32),
            jax.ShapeDtypeStruct((_NC * _N, _DH), jnp.float32),
            jax.ShapeDtypeStruct((_NC * _N, _DH), jnp.float32),
        ),
        mesh=mesh,
        compiler_params=pltpu.CompilerParams(use_tc_tiling_on_sc=False),
        scratch_types=[
            pltpu.VMEM((_CPT, _CHUNK), jnp.int32),
            pltpu.VMEM((_CPT, _CHUNK), jnp.int32),
            pltpu.VMEM((_CHUNK, _DH), jnp.float32),
            pltpu.VMEM((_CHUNK, _DH), jnp.float32),
            pltpu.VMEM((_CHUNK, _DH), jnp.float32),
            pltpu.VMEM((_CHUNK, _DH), jnp.float32),
            pltpu.VMEM((_CHUNK, _CW), jnp.float32),
            pltpu.VMEM_SHARED((_N, _DH), jnp.float32),
            pltpu.VMEM_SHARED((_N, _CW), jnp.float32),
            pltpu.SemaphoreType.DMA,
            pltpu.SemaphoreType.DMA,
            pltpu.SemaphoreType.DMA,
            pltpu.SemaphoreType.DMA,
            pltpu.SemaphoreType.DMA,
            pltpu.SemaphoreType.DMA,
            pltpu.SemaphoreType.DMA,
            pltpu.SemaphoreType.DMA,
            pltpu.SemaphoreType.DMA,
        ],
    )
    def k(x_item_h, x_user_h, src_iu_h, dst_iu_h, src_ui_h, dst_ui_h,
          zrow_h, zcnt_h, ones_hh,
          out_su, out_cu, out_si, out_ci, tbl_iu_h, tbl_ui_h,
          srcv, dstv, b0, b1, b2, b3, onesv, acc, accc,
          sg0, sg1, sg2, sg3, ss0, ss1, ss2, ss3, sc):
        cid = lax.axis_index("c")
        sid = lax.axis_index("s")
        r0 = sid * _RPS
        bufs = ((b0, sg0, ss0), (b1, sg1, ss1), (b2, sg2, ss2), (b3, sg3, ss3))

        pltpu.sync_copy(ones_hh, onesv)

        def zero_acc():
            pltpu.sync_copy(zrow_h, acc.at[pl.ds(r0, _RPS)])
            pltpu.sync_copy(zcnt_h, accc.at[pl.ds(r0, _RPS)])

        def stack_tables():
            jobs = []
            for kk in range(_RPS // _CHUNK):
                rr = r0 + kk * _CHUNK
                jobs.append((x_item_h, tbl_iu_h, rr))
                jobs.append((x_user_h, tbl_ui_h, rr))

            def rd(n):
                xh, _, rr = jobs[n]
                return xh.at[pl.ds(rr, _CHUNK), pl.ds(cid * _DH, _DH)]

            for n in range(min(_NBUF, len(jobs))):
                pltpu.async_copy(rd(n), bufs[n][0], bufs[n][1])
            for n in range(len(jobs)):
                buf, sg, _ = bufs[n % _NBUF]
                pltpu.make_async_copy(rd(n), buf, sg).wait()
                _, dsth, rr = jobs[n]
                pltpu.sync_copy(buf, dsth.at[pl.ds(cid * _N + rr, _CHUNK)])
                if n + _NBUF < len(jobs):
                    pltpu.async_copy(rd(n + _NBUF), buf, sg)

        def run_type(src_h, dst_h, table_h, out_s, out_c, cnt_core):
            pltpu.sync_copy(src_h.at[cid, sid], srcv)
            pltpu.sync_copy(dst_h.at[sid], dstv)
            do_cnt = cid == cnt_core

            def fire_gather(t, buf, sem):
                pltpu.async_copy(table_h.at[srcv.at[t]], buf, sem)

            def wait_gather(t, buf, sem):
                pltpu.make_async_copy(table_h.at[srcv.at[t]], buf, sem).wait()

            def fire_scatter(t, buf, sem):
                pltpu.async_copy(buf, acc.at[dstv.at[t]], sem, add=True)

                @pl.when(do_cnt)
                def _():
                    pltpu.async_copy(onesv, accc.at[dstv.at[t]], sc, add=True)

            def wait_scatter(t, buf, sem):
                pltpu.make_async_copy(buf, acc.at[dstv.at[t]], sem).wait()

                @pl.when(do_cnt)
                def _():
                    pltpu.make_async_copy(onesv, accc.at[dstv.at[t]], sc).wait()

            fire_gather(0, b0, sg0)
            fire_gather(1, b1, sg1)

            def body(i, carry):
                for kk in range(_NBUF):
                    t = _NBUF * i + kk
                    buf, sg, ss = bufs[kk]
                    rbuf, rsg, rss = bufs[(kk + 2) % _NBUF]
                    if kk >= 2:
                        wait_scatter(t - 2, rbuf, rss)
                    else:
                        @pl.when(t >= 2)
                        def _():
                            wait_scatter(t - 2, rbuf, rss)

                    @pl.when(t + 2 < _CPT)
                    def _():
                        fire_gather(t + 2, rbuf, rsg)

                    wait_gather(t, buf, sg)
                    fire_scatter(t, buf, ss)
                return carry

            lax.fori_loop(0, _CPT // _NBUF, body, 0)
            wait_scatter(_CPT - 2, bufs[(_CPT - 2) % _NBUF][0],
                         bufs[(_CPT - 2) % _NBUF][2])
            wait_scatter(_CPT - 1, bufs[(_CPT - 1) % _NBUF][0],
                         bufs[(_CPT - 1) % _NBUF][2])
            plsc.subcore_barrier()
            pltpu.sync_copy(acc.at[pl.ds(r0, _RPS)],
                            out_s.at[cid, pl.ds(r0, _RPS)])

            @pl.when(do_cnt)
            def _():
                pltpu.sync_copy(accc.at[pl.ds(r0, _RPS)],
                                out_c.at[pl.ds(r0, _RPS)])

        zero_acc()
        stack_tables()
        plsc.subcore_barrier()
        run_type(src_iu_h, dst_iu_h, tbl_iu_h, out_su, out_cu, 0)
        zero_acc()
        plsc.subcore_barrier()
        run_type(src_ui_h, dst_ui_h, tbl_ui_h, out_si, out_ci, 1)

    return k(x_item, x_user, src_iu, dst_iu, src_ui, dst_ui,
             zrow, zcnt, ones_h)[:4]


def _dense_body(xu, xi, su, cu, si, ci,
                wmiu, bmiu, wmui, bmui, wp1, bp1, wp2, bp2,
                wsu, bsu, wsi, bsi, wcu, bcu, wci, bci,
                out_u, out_i):
    f32 = jnp.float32
    cuv = cu[:, 0:1]
    gu = jnp.concatenate([su[0], su[1]], -1) / jnp.maximum(cuv, 1.0)
    mu = (cuv > 0.0).astype(f32)
    xuv = xu[...]
    h = jnp.maximum(jnp.dot(xuv, wp1[...]) + bp1[...], 0.0)
    pred = jnp.dot(h, wp2[...]) + bp2[...]
    aggstd = jnp.dot(gu, wmiu[...]) + bmiu[...] * mu
    neigh = 0.5 * (aggstd + gu - pred * mu)
    selfu = jnp.dot(xuv, wsu[...]) + bsu[...]
    wcu_t = wcu[...]
    out_u[...] = jnp.maximum(
        jnp.dot(selfu, wcu_t[:_D]) + jnp.dot(neigh, wcu_t[_D:]) + bcu[...], 0.0)
    civ = ci[:, 0:1]
    gi = jnp.concatenate([si[0], si[1]], -1) / jnp.maximum(civ, 1.0)
    mi = (civ > 0.0).astype(f32)
    xiv = xi[...]
    aggi = jnp.dot(gi, wmui[...]) + bmui[...] * mi
    selfi = jnp.dot(xiv, wsi[...]) + bsi[...]
    wci_t = wci[...]
    out_i[...] = jnp.maximum(
        jnp.dot(selfi, wci_t[:_D]) + jnp.dot(aggi, wci_t[_D:]) + bci[...], 0.0)


def _dense_combine(xu, xi, su, cu, si, ci,
                   wmiu, bmiu, wmui, bmui, wp1, bp1, wp2, bp2,
                   wsu, bsu, wsi, bsi, wcu, bcu, wci, bci):
    grid = _N // _TILE_ROWS
    row = lambda width: pl.BlockSpec((_TILE_ROWS, width), lambda i: (i, 0))
    half = pl.BlockSpec((_NC, _TILE_ROWS, _DH), lambda i: (0, i, 0))
    full = lambda a, b: pl.BlockSpec((a, b), lambda i: (0, 0))
    return pl.pallas_call(
        _dense_body,
        grid=(grid,),
        in_specs=[
            row(_D), row(_D),
            half, row(_CW),
            half, row(_CW),
            full(_D, _D), full(1, _D),
            full(_D, _D), full(1, _D),
            full(_D, _H), full(1, _H),
            full(_H, _D), full(1, _D),
            full(_D, _D), full(1, _D),
            full(_D, _D), full(1, _D),
            full(2 * _D, _D), full(1, _D),
            full(2 * _D, _D), full(1, _D),
        ],
        out_specs=[row(_D), row(_D)],
        out_shape=[
            jax.ShapeDtypeStruct((_N, _D), jnp.float32),
            jax.ShapeDtypeStruct((_N, _D), jnp.float32),
        ],
    )(xu, xi, su, cu, si, ci,
      wmiu, bmiu, wmui, bmui, wp1, bp1, wp2, bp2,
      wsu, bsu, wsi, bsi, wcu, bcu, wci, bci)


def _prep_edges(ei):
    src = ei[0].astype(jnp.int32)
    dst = ei[1].astype(jnp.int32)
    src2 = jnp.stack([src, src + _N]).reshape(_NC, _NS, _CPT, _CHUNK)
    return src2, dst.reshape(_NS, _CPT, _CHUNK)


def kernel(x_user, x_item, edge_index_item_to_user, edge_index_user_rev_item,
           W_msg_iu, b_msg_iu, W_msg_ui, b_msg_ui,
           W_pred1, b_pred1, W_pred2, b_pred2,
           W_self_user, b_self_user, W_self_item, b_self_item,
           W_comb_user, b_comb_user, W_comb_item, b_comb_item):
    x_user = x_user.astype(jnp.float32)
    x_item = x_item.astype(jnp.float32)
    src_iu, dst_iu = _prep_edges(edge_index_item_to_user)
    src_ui, dst_ui = _prep_edges(edge_index_user_rev_item)

    zrow = jnp.zeros((_RPS, _DH), jnp.float32)
    zcnt = jnp.zeros((_RPS, _CW), jnp.float32)
    ones_h = jnp.ones((_CHUNK, _CW), jnp.float32)

    su, cu, si, ci = _sc_segment_sums(
        x_item, x_user, src_iu, dst_iu, src_ui, dst_ui, zrow, zcnt, ones_h)

    out_u, out_i = _dense_combine(
        x_user, x_item, su, cu, si, ci,
        W_msg_iu.T, b_msg_iu.reshape(1, _D),
        W_msg_ui.T, b_msg_ui.reshape(1, _D),
        W_pred1.T, b_pred1.reshape(1, _H),
        W_pred2.T, b_pred2.reshape(1, _D),
        W_self_user.T, b_self_user.reshape(1, _D),
        W_self_item.T, b_self_item.reshape(1, _D),
        W_comb_user.T, b_comb_user.reshape(1, _D),
        W_comb_item.T, b_comb_item.reshape(1, _D))
    return (out_u, out_i)
